# baseline (speedup 1.0000x reference)
import functools

import jax
import jax.numpy as jnp
from jax import lax
from jax.experimental import pallas as pl
from jax.experimental.pallas import tpu as pltpu
from jax.experimental.pallas import tpu_sc as plsc

_BETA = 0.25

_SC_INFO = plsc.get_sparse_core_info()
_NC = _SC_INFO.num_cores
_NS = _SC_INFO.num_subcores
_NW = _NC * _NS


def _argmin_kernel(flat_ref, cb_ref, idx_ref, grp_ref):
    flat = flat_ref[...]
    cb = cb_ref[...]
    bn = flat.shape[0]
    k = cb.shape[0]

    mm2 = jax.lax.dot_general(
        flat * 2.0, cb,
        dimension_numbers=(((1,), (1,)), ((), ())),
        preferred_element_type=jnp.float32,
    )
    zsq = jnp.sum(flat * flat, axis=1, keepdims=True)
    c2 = jnp.sum(cb * cb, axis=1)[None, :]
    dist = (zsq + c2) - mm2

    minval = jnp.min(dist, axis=1, keepdims=True)
    lane = jax.lax.broadcasted_iota(jnp.int32, (bn, k), 1)
    idx = jnp.min(jnp.where(dist == minval, lane, k), axis=1, keepdims=True)
    idx_ref[...] = idx
    grp_ref[...] = idx >> 2


def _st_loss_kernel(flat_ref, q4_ref, idx_ref, qst_ref, part_ref):
    flat = flat_ref[...]
    q4 = q4_ref[...]
    sel = idx_ref[...] & 3
    d = flat.shape[1]
    q = q4[:, 0:d]
    for c in (1, 2, 3):
        q = jnp.where(sel == c, q4[:, c * d:(c + 1) * d], q)
    qst_ref[...] = flat + (q - flat)
    diff = q - flat
    part_ref[...] = jnp.sum(diff * diff).reshape(1, 1)


def _make_sc_gather(b, row, b_per_w):
    mesh = plsc.VectorSubcoreMesh(core_axis_name="c", subcore_axis_name="s")

    @functools.partial(
        pl.kernel, mesh=mesh,
        out_type=jax.ShapeDtypeStruct((b, row), jnp.float32),
        scratch_types=[
            pltpu.VMEM((b_per_w,), jnp.int32),
            pltpu.VMEM((b_per_w, row), jnp.float32),
            pltpu.SemaphoreType.DMA,
        ],
    )
    def _sc_gather(cbg_hbm, grp_hbm, out_hbm, grp_v, rows_v, sem):
        wid = lax.axis_index("s") * _NC + lax.axis_index("c")
        base = wid * b_per_w
        pltpu.sync_copy(grp_hbm.at[pl.ds(base, b_per_w)], grp_v)
        pltpu.async_copy(cbg_hbm.at[grp_v], rows_v, sem).wait()
        pltpu.sync_copy(rows_v, out_hbm.at[pl.ds(base, b_per_w)])

    return _sc_gather


@jax.jit
def kernel(data, codebook):
    orig_shape = data.shape
    d = data.shape[-1]
    flat = data.reshape(-1, d)
    n = flat.shape[0]
    k = codebook.shape[0]

    bn = 256
    n_blocks = n // bn
    inv_count = 1.0 / float(data.size)

    idx, grp = pl.pallas_call(
        _argmin_kernel,
        grid=(n_blocks,),
        in_specs=[
            pl.BlockSpec((bn, d), lambda i: (i, 0)),
            pl.BlockSpec((k, d), lambda i: (0, 0)),
        ],
        out_specs=[
            pl.BlockSpec((bn, 1), lambda i: (i, 0)),
            pl.BlockSpec((bn, 1), lambda i: (i, 0)),
        ],
        out_shape=[
            jax.ShapeDtypeStruct((n, 1), jnp.int32),
            jax.ShapeDtypeStruct((n, 1), jnp.int32),
        ],
        compiler_params=pltpu.CompilerParams(
            dimension_semantics=("arbitrary",),
        ),
    )(flat, codebook)

    cb_groups = codebook.reshape(k // 4, 4 * d)
    q4 = _make_sc_gather(n, 4 * d, n // _NW)(cb_groups, grp.reshape(n))

    qst, part = pl.pallas_call(
        _st_loss_kernel,
        in_specs=[
            pl.BlockSpec((n, d), lambda: (0, 0)),
            pl.BlockSpec((n, 4 * d), lambda: (0, 0)),
            pl.BlockSpec((n, 1), lambda: (0, 0)),
        ],
        out_specs=[
            pl.BlockSpec((n, d), lambda: (0, 0)),
            pl.BlockSpec((1, 1), lambda: (0, 0)),
        ],
        out_shape=[
            jax.ShapeDtypeStruct((n, d), jnp.float32),
            jax.ShapeDtypeStruct((1, 1), jnp.float32),
        ],
    )(flat, q4, idx)

    vq_loss = part[0, 0] * ((1.0 + _BETA) * inv_count)
    return qst.reshape(orig_shape), vq_loss


# speedup vs baseline: 1.0496x; 1.0496x over previous
import functools

import jax
import jax.numpy as jnp
from jax import lax
from jax.experimental import pallas as pl
from jax.experimental.pallas import tpu as pltpu
from jax.experimental.pallas import tpu_sc as plsc

_BETA = 0.25

_SC_INFO = plsc.get_sparse_core_info()
_NC = _SC_INFO.num_cores
_NS = _SC_INFO.num_subcores
_NW = _NC * _NS


def _argmin_kernel(flat_ref, cb_ref, idx_ref, grp_ref):
    flat = flat_ref[...]
    cb = cb_ref[...]
    bn = flat.shape[0]
    k = cb.shape[0]

    mm2 = jax.lax.dot_general(
        flat * 2.0, cb,
        dimension_numbers=(((1,), (1,)), ((), ())),
        preferred_element_type=jnp.float32,
    )
    zsq = jnp.sum(flat * flat, axis=1, keepdims=True)
    c2 = jnp.sum(cb * cb, axis=1)[None, :]
    dist = (zsq + c2) - mm2

    minval = jnp.min(dist, axis=1, keepdims=True)
    lane = jax.lax.broadcasted_iota(jnp.int32, (bn, k), 1)
    idx = jnp.min(jnp.where(dist == minval, lane, k), axis=1, keepdims=True)
    idx_ref[...] = idx
    grp_ref[...] = idx >> 2


def _st_loss_kernel(flat_ref, q4_ref, idx_ref, qst_ref, part_ref):
    flat = flat_ref[...]
    q4 = q4_ref[...]
    sel = idx_ref[...] & 3
    d = flat.shape[1]
    q = q4[:, 0:d]
    for c in (1, 2, 3):
        q = jnp.where(sel == c, q4[:, c * d:(c + 1) * d], q)
    qst_ref[...] = flat + (q - flat)
    diff = q - flat
    part_ref[...] = jnp.sum(diff * diff).reshape(1, 1)


def _make_sc_gather(b, row, b_per_w):
    mesh = plsc.VectorSubcoreMesh(core_axis_name="c", subcore_axis_name="s")

    @functools.partial(
        pl.kernel, mesh=mesh,
        out_type=jax.ShapeDtypeStruct((b, row), jnp.float32),
        scratch_types=[
            pltpu.VMEM((b_per_w,), jnp.int32),
            pltpu.VMEM((b_per_w, row), jnp.float32),
            pltpu.SemaphoreType.DMA,
        ],
    )
    def _sc_gather(cbg_hbm, grp_hbm, out_hbm, grp_v, rows_v, sem):
        wid = lax.axis_index("s") * _NC + lax.axis_index("c")
        base = wid * b_per_w
        pltpu.sync_copy(grp_hbm.at[pl.ds(base, b_per_w)], grp_v)
        pltpu.async_copy(cbg_hbm.at[grp_v], rows_v, sem).wait()
        pltpu.sync_copy(rows_v, out_hbm.at[pl.ds(base, b_per_w)])

    return _sc_gather


@jax.jit
def kernel(data, codebook):
    orig_shape = data.shape
    d = data.shape[-1]
    flat = data.reshape(-1, d)
    n = flat.shape[0]
    k = codebook.shape[0]

    bn = 1024
    n_blocks = n // bn
    inv_count = 1.0 / float(data.size)

    idx, grp = pl.pallas_call(
        _argmin_kernel,
        grid=(n_blocks,),
        in_specs=[
            pl.BlockSpec((bn, d), lambda i: (i, 0)),
            pl.BlockSpec((k, d), lambda i: (0, 0)),
        ],
        out_specs=[
            pl.BlockSpec((bn, 1), lambda i: (i, 0)),
            pl.BlockSpec((bn, 1), lambda i: (i, 0)),
        ],
        out_shape=[
            jax.ShapeDtypeStruct((n, 1), jnp.int32),
            jax.ShapeDtypeStruct((n, 1), jnp.int32),
        ],
        compiler_params=pltpu.CompilerParams(
            dimension_semantics=("arbitrary",),
        ),
    )(flat, codebook)

    cb_groups = codebook.reshape(k // 4, 4 * d)
    q4 = _make_sc_gather(n, 4 * d, n // _NW)(cb_groups, grp.reshape(n))

    qst, part = pl.pallas_call(
        _st_loss_kernel,
        in_specs=[
            pl.BlockSpec((n, d), lambda: (0, 0)),
            pl.BlockSpec((n, 4 * d), lambda: (0, 0)),
            pl.BlockSpec((n, 1), lambda: (0, 0)),
        ],
        out_specs=[
            pl.BlockSpec((n, d), lambda: (0, 0)),
            pl.BlockSpec((1, 1), lambda: (0, 0)),
        ],
        out_shape=[
            jax.ShapeDtypeStruct((n, d), jnp.float32),
            jax.ShapeDtypeStruct((1, 1), jnp.float32),
        ],
    )(flat, q4, idx)

    vq_loss = part[0, 0] * ((1.0 + _BETA) * inv_count)
    return qst.reshape(orig_shape), vq_loss
